# Initial kernel scaffold; baseline (speedup 1.0000x reference)
import jax
import jax.numpy as jnp
from jax.experimental import pallas as pl


def kernel(adj, input, fc_W, fc_b, a):
    raise NotImplementedError("write your pallas kernel here")


# speedup vs baseline: 8.2724x; 8.2724x over previous
import functools

import jax
import jax.numpy as jnp
from jax import lax
from jax.experimental import pallas as pl
from jax.experimental.pallas import tpu as pltpu, tpu_sc as plsc

N = 10000
E = 320000
D = 128
ALPHA = 0.2

NC = 2
NS = 16
NW = NC * NS
EPT = E // NW
CH = 80
NCHUNK = EPT // CH
NPAD = 10240
SLICE = NPAD // NS

_mesh = plsc.VectorSubcoreMesh(
    core_axis_name="c", subcore_axis_name="s", num_cores=NC, num_subcores=NS)


def _dense_body(x_ref, w_ref, b_ref, a1_ref, a2_ref, h_ref, s1_ref, s2_ref):
    h = lax.dot_general(x_ref[...], w_ref[...], (((1,), (1,)), ((), ())),
                        preferred_element_type=jnp.float32)
    h = h + b_ref[...]
    h_ref[...] = h
    s1_ref[...] = jnp.dot(h, a1_ref[...], preferred_element_type=jnp.float32)
    s2_ref[...] = jnp.dot(h, a2_ref[...], preferred_element_type=jnp.float32)


def _dense(x, w, b2, a1, a2):
    blk = 1000
    grid = N // blk
    return pl.pallas_call(
        _dense_body,
        grid=(grid,),
        in_specs=[
            pl.BlockSpec((blk, D), lambda i: (i, 0)),
            pl.BlockSpec((D, D), lambda i: (0, 0)),
            pl.BlockSpec((1, D), lambda i: (0, 0)),
            pl.BlockSpec((D, 1), lambda i: (0, 0)),
            pl.BlockSpec((D, 1), lambda i: (0, 0)),
        ],
        out_specs=[
            pl.BlockSpec((blk, D), lambda i: (i, 0)),
            pl.BlockSpec((blk, 1), lambda i: (i, 0)),
            pl.BlockSpec((blk, 1), lambda i: (i, 0)),
        ],
        out_shape=[
            jax.ShapeDtypeStruct((N, D), jnp.float32),
            jax.ShapeDtypeStruct((N, 1), jnp.float32),
            jax.ShapeDtypeStruct((N, 1), jnp.float32),
        ],
    )(x, w, b2, a1, a2)


def _edge_body(srcf, dstf, s1h, s2h, p_out, den_out,
               src_v, dst_v, s1_v, s2_v, p_v, den_v, tmp_v, acc_v, shr):
    c = lax.axis_index("c")
    s = lax.axis_index("s")
    wid = c * NS + s
    pltpu.sync_copy(srcf.at[pl.ds(wid * EPT, EPT)], src_v)
    pltpu.sync_copy(dstf.at[pl.ds(wid * EPT, EPT)], dst_v)
    pltpu.sync_copy(s1h, s1_v)
    pltpu.sync_copy(s2h, s2_v)

    zero16 = jnp.zeros((16,), jnp.float32)

    def _zero(i, carry):
        den_v[pl.ds(i * 16, 16)] = zero16
        return carry
    lax.fori_loop(0, NPAD // 16, _zero, 0)

    def _step(i, carry):
        sl = pl.ds(i * 16, 16)
        si = src_v[sl]
        di = dst_v[sl]
        e = plsc.load_gather(s1_v, [si]) + plsc.load_gather(s2_v, [di])
        e = jnp.where(e > 0, e, e * ALPHA)
        p = jnp.exp(e)
        p_v[sl] = p
        plsc.addupdate_scatter(den_v, [di], p)
        return carry
    lax.fori_loop(0, EPT // 16, _step, 0)

    pltpu.sync_copy(p_v, p_out.at[pl.ds(wid * EPT, EPT)])

    pltpu.sync_copy(den_v, shr.at[s])
    plsc.subcore_barrier()
    base = s * SLICE

    def _zacc(i, carry):
        acc_v[pl.ds(i * 16, 16)] = zero16
        return carry
    lax.fori_loop(0, SLICE // 16, _zacc, 0)

    def _red(t, carry):
        pltpu.sync_copy(shr.at[t, pl.ds(base, SLICE)], tmp_v)

        def _add(i, c2):
            sl = pl.ds(i * 16, 16)
            acc_v[sl] = acc_v[sl] + tmp_v[sl]
            return c2
        lax.fori_loop(0, SLICE // 16, _add, 0)
        return carry
    lax.fori_loop(0, NS, _red, 0)
    pltpu.sync_copy(acc_v, den_out.at[pl.ds(c * NPAD + base, SLICE)])


_edge_logits = functools.partial(
    pl.kernel,
    out_type=[
        jax.ShapeDtypeStruct((E,), jnp.float32),
        jax.ShapeDtypeStruct((NC * NPAD,), jnp.float32),
    ],
    mesh=_mesh,
    compiler_params=pltpu.CompilerParams(needs_layout_passes=False),
    scratch_types=[
        pltpu.VMEM((EPT,), jnp.int32),
        pltpu.VMEM((EPT,), jnp.int32),
        pltpu.VMEM((N,), jnp.float32),
        pltpu.VMEM((N,), jnp.float32),
        pltpu.VMEM((EPT,), jnp.float32),
        pltpu.VMEM((NPAD,), jnp.float32),
        pltpu.VMEM((SLICE,), jnp.float32),
        pltpu.VMEM((SLICE,), jnp.float32),
        pltpu.VMEM_SHARED((NS, NPAD), jnp.float32),
    ],
)(_edge_body)


def _agg_body(srcc, dstc, ph, denh, hh, op,
              den_v, tmp_v, p_c, srcv, dstv, rows, wv, zrows, acc, sem):
    c = lax.axis_index("c")
    s = lax.axis_index("s")
    wid = c * NS + s

    pltpu.sync_copy(denh.at[pl.ds(0, NPAD)], den_v)
    pltpu.sync_copy(denh.at[pl.ds(NPAD, NPAD)], tmp_v)

    def _cmb(i, carry):
        sl = pl.ds(i * 16, 16)
        den_v[sl] = den_v[sl] + tmp_v[sl]
        return carry
    lax.fori_loop(0, NPAD // 16, _cmb, 0)

    iotas = [lax.iota(jnp.int32, 16) + 16 * k for k in range(8)]
    zero16 = jnp.zeros((16,), jnp.float32)

    def _zr(r, carry):
        r16 = jnp.full((16,), r, jnp.int32)
        for k in range(8):
            plsc.store_scatter(zrows, [r16, iotas[k]], zero16)
        return carry
    lax.fori_loop(0, 64, _zr, 0)
    base = s * SLICE

    def _zacc(r, carry):
        pltpu.sync_copy(zrows, acc.at[pl.ds(base + r * 64, 64), :])
        return carry
    lax.fori_loop(0, SLICE // 64, _zacc, 0)
    plsc.subcore_barrier()

    def _chunk(j, carry):
        pltpu.sync_copy(srcc.at[pl.ds(wid * EPT + j * CH, CH)], srcv)
        pltpu.sync_copy(dstc.at[pl.ds(wid * EPT + j * CH, CH)], dstv)
        pltpu.sync_copy(ph.at[pl.ds(wid * EPT + j * CH, CH)], p_c)
        pltpu.async_copy(hh.at[dstv], rows, sem).wait()

        def _wg(g, c2):
            sl = pl.ds(g * 16, 16)
            d16 = plsc.load_gather(den_v, [dstv[sl]])
            p16 = p_c[sl]
            wv[sl] = p16 / d16
            return c2
        lax.fori_loop(0, CH // 16, _wg, 0)

        def _scale(e, c2):
            e16 = jnp.full((16,), e, jnp.int32)
            w16 = plsc.load_gather(wv, [e16])
            for k in range(8):
                v = plsc.load_gather(rows, [e16, iotas[k]])
                plsc.store_scatter(rows, [e16, iotas[k]], v * w16)
            return c2
        lax.fori_loop(0, CH, _scale, 0)

        pltpu.sync_copy(rows, acc.at[srcv], add=True)
        return carry
    lax.fori_loop(0, NCHUNK, _chunk, 0)
    plsc.subcore_barrier()

    def _wb(r, carry):
        pltpu.sync_copy(acc.at[pl.ds(base + r * 64, 64), :],
                        op.at[c, pl.ds(base + r * 64, 64), :])
        return carry
    lax.fori_loop(0, SLICE // 64, _wb, 0)


_aggregate = functools.partial(
    pl.kernel,
    out_type=jax.ShapeDtypeStruct((NC, NPAD, D), jnp.float32),
    mesh=_mesh,
    compiler_params=pltpu.CompilerParams(needs_layout_passes=False),
    scratch_types=[
        pltpu.VMEM((NPAD,), jnp.float32),
        pltpu.VMEM((NPAD,), jnp.float32),
        pltpu.VMEM((CH,), jnp.float32),
        pltpu.VMEM((CH,), jnp.int32),
        pltpu.VMEM((CH,), jnp.int32),
        pltpu.VMEM((CH, D), jnp.float32),
        pltpu.VMEM((CH,), jnp.float32),
        pltpu.VMEM((64, D), jnp.float32),
        pltpu.VMEM_SHARED((NPAD, D), jnp.float32),
        pltpu.SemaphoreType.DMA,
    ],
)(_agg_body)


def _fin_body(p0_ref, p1_ref, o_ref):
    x = p0_ref[...] + p1_ref[...]
    o_ref[...] = jnp.where(x > 0, x, jnp.exp(x) - 1.0)


def _finish(p0, p1):
    blk = 1000
    return pl.pallas_call(
        _fin_body,
        grid=(N // blk,),
        in_specs=[
            pl.BlockSpec((blk, D), lambda i: (i, 0)),
            pl.BlockSpec((blk, D), lambda i: (i, 0)),
        ],
        out_specs=pl.BlockSpec((blk, D), lambda i: (i, 0)),
        out_shape=jax.ShapeDtypeStruct((N, D), jnp.float32),
    )(p0, p1)


def kernel(adj, input, fc_W, fc_b, a):
    adj = adj.astype(jnp.int32)
    h, s1, s2 = _dense(input, fc_W, fc_b.reshape(1, D),
                       a[0, :D].reshape(D, 1), a[0, D:].reshape(D, 1))
    src = adj[0]
    dst = adj[1]
    p, den = _edge_logits(src, dst, s1.reshape(N), s2.reshape(N))
    parts = _aggregate(src, dst, p, den, h)
    return _finish(parts[0, :N], parts[1, :N])
